# Initial kernel scaffold; baseline (speedup 1.0000x reference)
import jax
import jax.numpy as jnp
from jax.experimental import pallas as pl


def kernel(x, edge_index, W_in, b_in, Wl0, bl0, Wr0, Wl1, bl1, Wr1, Wl2, bl2, Wr2, Wl3, bl3, Wr3):
    raise NotImplementedError("write your pallas kernel here")


# speedup vs baseline: 3.3097x; 3.3097x over previous
import functools

import jax
import jax.numpy as jnp
from jax import lax
from jax.experimental import pallas as pl
from jax.experimental.pallas import tpu as pltpu
from jax.experimental.pallas import tpu_sc as plsc

NN = 10000
EE = 320000
D = 128
NTILES = 16
NSC = 2
CHUNK = 128
CW = 128
G = 8
CHUNKS = (-(-EE // (NSC * NTILES * CHUNK * G))) * G
GROUPS = CHUNKS // G
EPT = CHUNKS * CHUNK
EPAD = NSC * NTILES * EPT
NPAD = (-(-(NN + 1) // (NTILES * CHUNK))) * (NTILES * CHUNK)
RPT = NPAD // NTILES
BR = 1000
GRID = NN // BR

_mesh = plsc.VectorSubcoreMesh(core_axis_name="c", subcore_axis_name="s")


@functools.partial(
    pl.kernel,
    mesh=_mesh,
    out_type=jax.ShapeDtypeStruct((NSC, NPAD, D), jnp.float32),
    scratch_types=[
        pltpu.VMEM((G, CHUNK), jnp.int32),
        pltpu.VMEM((G, CHUNK), jnp.int32),
        pltpu.VMEM((CHUNK, D), jnp.float32),
        pltpu.VMEM_SHARED((NPAD, D), jnp.float32),
        pltpu.SemaphoreType.DMA,
    ],
)
def _sc_agg(h_hbm, src_hbm, dst_hbm, out_hbm, src_g, dst_g, rows_v,
            acc_sh, gsem):
    c = lax.axis_index("c")
    s = lax.axis_index("s")

    z = jnp.zeros((16,), jnp.float32)

    def _zrow(r, carry):
        def _zcol(j, carry2):
            rows_v[r, pl.ds(j * 16, 16)] = z
            return carry2
        return lax.fori_loop(0, D // 16, _zcol, carry)
    lax.fori_loop(0, CHUNK, _zrow, 0)
    for k in range(RPT // CHUNK):
        pltpu.sync_copy(rows_v, acc_sh.at[pl.ds(s * RPT + k * CHUNK, CHUNK)])
    plsc.subcore_barrier()

    def _group(q, carry):
        pltpu.sync_copy(src_hbm.at[c, s, pl.ds(q * G, G)], src_g)
        pltpu.sync_copy(dst_hbm.at[c, s, pl.ds(q * G, G)], dst_g)
        for b in range(G):
            pltpu.async_copy(h_hbm.at[src_g.at[b]], rows_v, gsem).wait()
            pltpu.sync_copy(rows_v, acc_sh.at[dst_g.at[b]], add=True)
        return carry
    lax.fori_loop(0, GROUPS, _group, 0)
    plsc.subcore_barrier()

    pltpu.sync_copy(acc_sh.at[pl.ds(s * RPT, RPT)],
                    out_hbm.at[c, pl.ds(s * RPT, RPT)])


@functools.partial(
    pl.kernel,
    mesh=_mesh,
    out_type=jax.ShapeDtypeStruct((NSC, NPAD, CW), jnp.float32),
    scratch_types=[
        pltpu.VMEM((G, CHUNK), jnp.int32),
        pltpu.VMEM((CHUNK, CW), jnp.float32),
        pltpu.VMEM_SHARED((NPAD, CW), jnp.float32),
    ],
)
def _sc_cnt(dst_hbm, out_hbm, dst_g, ones_v, acc_sh):
    c = lax.axis_index("c")
    s = lax.axis_index("s")

    one = jnp.full((16,), 1.0, jnp.float32)
    z = jnp.zeros((16,), jnp.float32)

    def _fill(val):
        def _row(r, carry):
            def _col(j, carry2):
                ones_v[r, pl.ds(j * 16, 16)] = val
                return carry2
            return lax.fori_loop(0, CW // 16, _col, carry)
        lax.fori_loop(0, CHUNK, _row, 0)

    _fill(z)
    for k in range(RPT // CHUNK):
        pltpu.sync_copy(ones_v, acc_sh.at[pl.ds(s * RPT + k * CHUNK, CHUNK)])
    _fill(one)
    plsc.subcore_barrier()

    def _group(q, carry):
        pltpu.sync_copy(dst_hbm.at[c, s, pl.ds(q * G, G)], dst_g)
        for b in range(G):
            pltpu.sync_copy(ones_v, acc_sh.at[dst_g.at[b]], add=True)
        return carry
    lax.fori_loop(0, GROUPS, _group, 0)
    plsc.subcore_barrier()

    pltpu.sync_copy(acc_sh.at[pl.ds(s * RPT, RPT)],
                    out_hbm.at[c, pl.ds(s * RPT, RPT)])


def _proj_body(x_ref, wT_ref, b_ref, inp0_ref, h_ref):
    t = jnp.dot(x_ref[...], wT_ref[...], preferred_element_type=jnp.float32)
    t = t + b_ref[...]
    inp0_ref[...] = t
    h_ref[...] = jnp.maximum(t, 0.0)


_tc_proj = pl.pallas_call(
    _proj_body,
    grid=(GRID,),
    in_specs=[
        pl.BlockSpec((BR, D), lambda i: (i, 0)),
        pl.BlockSpec((D, D), lambda i: (0, 0)),
        pl.BlockSpec((1, D), lambda i: (0, 0)),
    ],
    out_specs=[pl.BlockSpec((BR, D), lambda i: (i, 0))] * 2,
    out_shape=[jax.ShapeDtypeStruct((NN, D), jnp.float32)] * 2,
)


def _layer_body(last, p_ref, c_ref, h_ref, inp0_ref, wlT_ref, bl_ref, wrT_ref,
                o_ref):
    p = p_ref[0] + p_ref[1]
    cnt = c_ref[0, :, 0:1] + c_ref[1, :, 0:1]
    mean = p / jnp.maximum(cnt, 1.0)
    acc = jnp.dot(mean, wlT_ref[...], preferred_element_type=jnp.float32)
    acc = acc + bl_ref[...]
    acc = acc + jnp.dot(h_ref[...], wrT_ref[...],
                        preferred_element_type=jnp.float32)
    if last:
        m = jnp.max(acc, axis=-1, keepdims=True)
        lse = jnp.log(jnp.sum(jnp.exp(acc - m), axis=-1, keepdims=True)) + m
        o_ref[...] = acc - lse
    else:
        o_ref[...] = jnp.maximum(acc, 0.0) + 0.2 * inp0_ref[...]


def _make_tc_layer(last):
    return pl.pallas_call(
        functools.partial(_layer_body, last),
        grid=(GRID,),
        in_specs=[
            pl.BlockSpec((NSC, BR, D), lambda i: (0, i, 0)),
            pl.BlockSpec((NSC, BR, CW), lambda i: (0, i, 0)),
            pl.BlockSpec((BR, D), lambda i: (i, 0)),
            pl.BlockSpec((BR, D), lambda i: (i, 0)),
            pl.BlockSpec((D, D), lambda i: (0, 0)),
            pl.BlockSpec((1, D), lambda i: (0, 0)),
            pl.BlockSpec((D, D), lambda i: (0, 0)),
        ],
        out_specs=pl.BlockSpec((BR, D), lambda i: (i, 0)),
        out_shape=jax.ShapeDtypeStruct((NN, D), jnp.float32),
    )


_tc_mid = _make_tc_layer(False)
_tc_last = _make_tc_layer(True)


def kernel(x, edge_index, W_in, b_in, Wl0, bl0, Wr0, Wl1, bl1, Wr1,
           Wl2, bl2, Wr2, Wl3, bl3, Wr3):
    pad = EPAD - EE
    srcp = jnp.concatenate(
        [edge_index[0], jnp.zeros((pad,), jnp.int32)]
    ).reshape(NSC, NTILES, CHUNKS, CHUNK)
    dstp = jnp.concatenate(
        [edge_index[1], jnp.full((pad,), NN, jnp.int32)]
    ).reshape(NSC, NTILES, CHUNKS, CHUNK)

    cntP = _sc_cnt(dstp)
    inp0, h = _tc_proj(x, W_in.T, b_in.reshape(1, D))
    layers = [(Wl0, bl0, Wr0), (Wl1, bl1, Wr1), (Wl2, bl2, Wr2),
              (Wl3, bl3, Wr3)]
    for i, (Wl, bl, Wr) in enumerate(layers):
        P = _sc_agg(h, srcp, dstp)
        tc = _tc_last if i == len(layers) - 1 else _tc_mid
        h = tc(P, cntP, h, inp0, Wl.T, bl.reshape(1, D), Wr.T)
    return h


# speedup vs baseline: 3.5584x; 1.0751x over previous
import functools

import jax
import jax.numpy as jnp
from jax import lax
from jax.experimental import pallas as pl
from jax.experimental.pallas import tpu as pltpu
from jax.experimental.pallas import tpu_sc as plsc

NN = 10000
EE = 320000
D = 128
NTILES = 16
NSC = 2
CHUNK = 128
CW = 128
G = 8
CHUNKS = (-(-EE // (NSC * NTILES * CHUNK * G))) * G
GROUPS = CHUNKS // G
EPT = CHUNKS * CHUNK
EPAD = NSC * NTILES * EPT
NPAD = (-(-(NN + 1) // (NTILES * CHUNK))) * (NTILES * CHUNK)
RPT = NPAD // NTILES
BR = 1000
GRID = NN // BR

_mesh = plsc.VectorSubcoreMesh(core_axis_name="c", subcore_axis_name="s")


@functools.partial(
    pl.kernel,
    mesh=_mesh,
    out_type=jax.ShapeDtypeStruct((NSC, NPAD, D), jnp.float32),
    scratch_types=[
        pltpu.VMEM((2, G, CHUNK), jnp.int32),
        pltpu.VMEM((2, G, CHUNK), jnp.int32),
        pltpu.VMEM((2, CHUNK, D), jnp.float32),
        pltpu.VMEM_SHARED((NPAD, D), jnp.float32),
        pltpu.SemaphoreType.DMA,
        pltpu.SemaphoreType.DMA,
        pltpu.SemaphoreType.DMA,
        pltpu.SemaphoreType.DMA,
        pltpu.SemaphoreType.DMA,
    ],
)
def _sc_agg(h_hbm, src_hbm, dst_hbm, out_hbm, src_g, dst_g, rows_v,
            acc_sh, gsem0, gsem1, ssem0, ssem1, isem):
    c = lax.axis_index("c")
    s = lax.axis_index("s")
    gsem = (gsem0, gsem1)
    ssem = (ssem0, ssem1)

    z = jnp.zeros((16,), jnp.float32)

    def _zrow(r, carry):
        def _zcol(j, carry2):
            rows_v[0, r, pl.ds(j * 16, 16)] = z
            return carry2
        return lax.fori_loop(0, D // 16, _zcol, carry)
    lax.fori_loop(0, CHUNK, _zrow, 0)
    for k in range(RPT // CHUNK):
        pltpu.sync_copy(rows_v.at[0],
                        acc_sh.at[pl.ds(s * RPT + k * CHUNK, CHUNK)])
    plsc.subcore_barrier()

    def _idx_start(g, slot):
        pltpu.async_copy(src_hbm.at[c, s, pl.ds(g * G, G)], src_g.at[slot],
                         isem)
        pltpu.async_copy(dst_hbm.at[c, s, pl.ds(g * G, G)], dst_g.at[slot],
                         isem)

    def _idx_drain(slot):
        pltpu.make_async_copy(src_hbm.at[c, s, pl.ds(0, G)], src_g.at[slot],
                              isem).wait()
        pltpu.make_async_copy(dst_hbm.at[c, s, pl.ds(0, G)], dst_g.at[slot],
                              isem).wait()

    _idx_start(0, 0)

    def _pair(m, carry):
        for qq in range(2):
            g = 2 * m + qq
            slot = qq
            _idx_drain(slot)
            _idx_start(lax.rem(g + 1, GROUPS), (qq + 1) % 2)
            gathers = [None] * G
            scatters = [None] * G
            gathers[0] = pltpu.async_copy(
                h_hbm.at[src_g.at[slot, 0]], rows_v.at[0], gsem[0])
            for b in range(G):
                if b + 1 < G:
                    if b >= 1:
                        scatters[b - 1].wait()
                    gathers[b + 1] = pltpu.async_copy(
                        h_hbm.at[src_g.at[slot, b + 1]],
                        rows_v.at[(b + 1) % 2], gsem[(b + 1) % 2])
                gathers[b].wait()
                scatters[b] = pltpu.async_copy(
                    rows_v.at[b % 2], acc_sh.at[dst_g.at[slot, b]],
                    ssem[b % 2], add=True)
            scatters[G - 2].wait()
            scatters[G - 1].wait()
        return carry
    lax.fori_loop(0, GROUPS // 2, _pair, 0)
    _idx_drain(0)
    plsc.subcore_barrier()

    pltpu.sync_copy(acc_sh.at[pl.ds(s * RPT, RPT)],
                    out_hbm.at[c, pl.ds(s * RPT, RPT)])


@functools.partial(
    pl.kernel,
    mesh=_mesh,
    out_type=jax.ShapeDtypeStruct((NSC, NPAD, CW), jnp.float32),
    scratch_types=[
        pltpu.VMEM((G, CHUNK), jnp.int32),
        pltpu.VMEM((CHUNK, CW), jnp.float32),
        pltpu.VMEM_SHARED((NPAD, CW), jnp.float32),
    ],
)
def _sc_cnt(dst_hbm, out_hbm, dst_g, ones_v, acc_sh):
    c = lax.axis_index("c")
    s = lax.axis_index("s")

    one = jnp.full((16,), 1.0, jnp.float32)
    z = jnp.zeros((16,), jnp.float32)

    def _fill(val):
        def _row(r, carry):
            def _col(j, carry2):
                ones_v[r, pl.ds(j * 16, 16)] = val
                return carry2
            return lax.fori_loop(0, CW // 16, _col, carry)
        lax.fori_loop(0, CHUNK, _row, 0)

    _fill(z)
    for k in range(RPT // CHUNK):
        pltpu.sync_copy(ones_v, acc_sh.at[pl.ds(s * RPT + k * CHUNK, CHUNK)])
    _fill(one)
    plsc.subcore_barrier()

    def _group(q, carry):
        pltpu.sync_copy(dst_hbm.at[c, s, pl.ds(q * G, G)], dst_g)
        for b in range(G):
            pltpu.sync_copy(ones_v, acc_sh.at[dst_g.at[b]], add=True)
        return carry
    lax.fori_loop(0, GROUPS, _group, 0)
    plsc.subcore_barrier()

    pltpu.sync_copy(acc_sh.at[pl.ds(s * RPT, RPT)],
                    out_hbm.at[c, pl.ds(s * RPT, RPT)])


def _proj_body(x_ref, wT_ref, b_ref, inp0_ref, h_ref):
    t = jnp.dot(x_ref[...], wT_ref[...], preferred_element_type=jnp.float32)
    t = t + b_ref[...]
    inp0_ref[...] = t
    h_ref[...] = jnp.maximum(t, 0.0)


_tc_proj = pl.pallas_call(
    _proj_body,
    grid=(GRID,),
    in_specs=[
        pl.BlockSpec((BR, D), lambda i: (i, 0)),
        pl.BlockSpec((D, D), lambda i: (0, 0)),
        pl.BlockSpec((1, D), lambda i: (0, 0)),
    ],
    out_specs=[pl.BlockSpec((BR, D), lambda i: (i, 0))] * 2,
    out_shape=[jax.ShapeDtypeStruct((NN, D), jnp.float32)] * 2,
)


def _layer_body(last, p_ref, c_ref, h_ref, inp0_ref, wlT_ref, bl_ref, wrT_ref,
                o_ref):
    p = p_ref[0] + p_ref[1]
    cnt = c_ref[0, :, 0:1] + c_ref[1, :, 0:1]
    mean = p / jnp.maximum(cnt, 1.0)
    acc = jnp.dot(mean, wlT_ref[...], preferred_element_type=jnp.float32)
    acc = acc + bl_ref[...]
    acc = acc + jnp.dot(h_ref[...], wrT_ref[...],
                        preferred_element_type=jnp.float32)
    if last:
        m = jnp.max(acc, axis=-1, keepdims=True)
        lse = jnp.log(jnp.sum(jnp.exp(acc - m), axis=-1, keepdims=True)) + m
        o_ref[...] = acc - lse
    else:
        o_ref[...] = jnp.maximum(acc, 0.0) + 0.2 * inp0_ref[...]


def _make_tc_layer(last):
    return pl.pallas_call(
        functools.partial(_layer_body, last),
        grid=(GRID,),
        in_specs=[
            pl.BlockSpec((NSC, BR, D), lambda i: (0, i, 0)),
            pl.BlockSpec((NSC, BR, CW), lambda i: (0, i, 0)),
            pl.BlockSpec((BR, D), lambda i: (i, 0)),
            pl.BlockSpec((BR, D), lambda i: (i, 0)),
            pl.BlockSpec((D, D), lambda i: (0, 0)),
            pl.BlockSpec((1, D), lambda i: (0, 0)),
            pl.BlockSpec((D, D), lambda i: (0, 0)),
        ],
        out_specs=pl.BlockSpec((BR, D), lambda i: (i, 0)),
        out_shape=jax.ShapeDtypeStruct((NN, D), jnp.float32),
    )


_tc_mid = _make_tc_layer(False)
_tc_last = _make_tc_layer(True)


def kernel(x, edge_index, W_in, b_in, Wl0, bl0, Wr0, Wl1, bl1, Wr1,
           Wl2, bl2, Wr2, Wl3, bl3, Wr3):
    pad = EPAD - EE
    srcp = jnp.concatenate(
        [edge_index[0], jnp.zeros((pad,), jnp.int32)]
    ).reshape(NSC, NTILES, CHUNKS, CHUNK)
    dstp = jnp.concatenate(
        [edge_index[1], jnp.full((pad,), NN, jnp.int32)]
    ).reshape(NSC, NTILES, CHUNKS, CHUNK)

    cntP = _sc_cnt(dstp)
    inp0, h = _tc_proj(x, W_in.T, b_in.reshape(1, D))
    layers = [(Wl0, bl0, Wr0), (Wl1, bl1, Wr1), (Wl2, bl2, Wr2),
              (Wl3, bl3, Wr3)]
    for i, (Wl, bl, Wr) in enumerate(layers):
        P = _sc_agg(h, srcp, dstp)
        tc = _tc_last if i == len(layers) - 1 else _tc_mid
        h = tc(P, cntP, h, inp0, Wl.T, bl.reshape(1, D), Wr.T)
    return h


# speedup vs baseline: 10.6675x; 2.9979x over previous
import functools

import jax
import jax.numpy as jnp
from jax import lax
from jax.experimental import pallas as pl
from jax.experimental.pallas import tpu as pltpu
from jax.experimental.pallas import tpu_sc as plsc

NN = 10000
EE = 320000
D = 128
NTILES = 16
NSC = 2
CHUNK = 128
CW = 128
G = 8
CHUNKS = (-(-EE // (NSC * NTILES * CHUNK * G))) * G
GROUPS = CHUNKS // G
EPT = CHUNKS * CHUNK
EPAD = NSC * NTILES * EPT
NPAD = (-(-(NN + 1) // (NTILES * CHUNK))) * (NTILES * CHUNK)
RPT = NPAD // NTILES
BR = 1000
GRID = NN // BR

_mesh = plsc.VectorSubcoreMesh(core_axis_name="c", subcore_axis_name="s")


@functools.partial(
    pl.kernel,
    mesh=_mesh,
    out_type=jax.ShapeDtypeStruct((NSC, NPAD, D), jnp.float32),
    scratch_types=[
        pltpu.VMEM((2, G, CHUNK), jnp.int32),
        pltpu.VMEM((2, G, CHUNK), jnp.int32),
        pltpu.VMEM((2, CHUNK, D), jnp.float32),
        pltpu.VMEM_SHARED((NPAD, D), jnp.float32),
        pltpu.SemaphoreType.DMA,
        pltpu.SemaphoreType.DMA,
        pltpu.SemaphoreType.DMA,
        pltpu.SemaphoreType.DMA,
        pltpu.SemaphoreType.DMA,
    ],
)
def _sc_agg(h_hbm, src_hbm, dst_hbm, out_hbm, src_g, dst_g, rows_v,
            acc_sh, gsem0, gsem1, ssem0, ssem1, isem):
    c = lax.axis_index("c")
    s = lax.axis_index("s")
    gsem = (gsem0, gsem1)
    ssem = (ssem0, ssem1)

    z = jnp.zeros((16,), jnp.float32)

    def _zrow(r, carry):
        def _zcol(j, carry2):
            rows_v[0, r, pl.ds(j * 16, 16)] = z
            return carry2
        return lax.fori_loop(0, D // 16, _zcol, carry)
    lax.fori_loop(0, CHUNK, _zrow, 0)
    for k in range(RPT // CHUNK):
        pltpu.sync_copy(rows_v.at[0],
                        acc_sh.at[pl.ds(s * RPT + k * CHUNK, CHUNK)])
    plsc.subcore_barrier()

    def _idx_start(g, slot):
        pltpu.async_copy(src_hbm.at[c, s, pl.ds(g * G, G)], src_g.at[slot],
                         isem)
        pltpu.async_copy(dst_hbm.at[c, s, pl.ds(g * G, G)], dst_g.at[slot],
                         isem)

    def _idx_drain(slot):
        pltpu.make_async_copy(src_hbm.at[c, s, pl.ds(0, G)], src_g.at[slot],
                              isem).wait()
        pltpu.make_async_copy(dst_hbm.at[c, s, pl.ds(0, G)], dst_g.at[slot],
                              isem).wait()

    _idx_start(0, 0)

    def _pair(m, carry):
        for qq in range(2):
            g = 2 * m + qq
            slot = qq
            _idx_drain(slot)
            _idx_start(lax.rem(g + 1, GROUPS), (qq + 1) % 2)
            gathers = [None] * G
            scatters = [None] * G
            gathers[0] = pltpu.async_copy(
                h_hbm.at[src_g.at[slot, 0]], rows_v.at[0], gsem[0])
            for b in range(G):
                if b + 1 < G:
                    if b >= 1:
                        scatters[b - 1].wait()
                    gathers[b + 1] = pltpu.async_copy(
                        h_hbm.at[src_g.at[slot, b + 1]],
                        rows_v.at[(b + 1) % 2], gsem[(b + 1) % 2])
                gathers[b].wait()
                scatters[b] = pltpu.async_copy(
                    rows_v.at[b % 2], acc_sh.at[dst_g.at[slot, b]],
                    ssem[b % 2], add=True)
            scatters[G - 2].wait()
            scatters[G - 1].wait()
        return carry
    lax.fori_loop(0, GROUPS // 2, _pair, 0)
    _idx_drain(0)
    plsc.subcore_barrier()

    pltpu.sync_copy(acc_sh.at[pl.ds(s * RPT, RPT)],
                    out_hbm.at[c, pl.ds(s * RPT, RPT)])


@functools.partial(
    pl.kernel,
    mesh=_mesh,
    out_type=jax.ShapeDtypeStruct((NSC, NPAD, CW), jnp.float32),
    scratch_types=[
        pltpu.VMEM((G, CHUNK), jnp.int32),
        pltpu.VMEM((CHUNK, CW), jnp.float32),
        pltpu.VMEM_SHARED((NPAD, CW), jnp.float32),
    ],
)
def _sc_cnt(dst_hbm, out_hbm, dst_g, ones_v, acc_sh):
    c = lax.axis_index("c")
    s = lax.axis_index("s")

    one = jnp.full((16,), 1.0, jnp.float32)
    z = jnp.zeros((16,), jnp.float32)

    def _fill(val):
        def _row(r, carry):
            def _col(j, carry2):
                ones_v[r, pl.ds(j * 16, 16)] = val
                return carry2
            return lax.fori_loop(0, CW // 16, _col, carry)
        lax.fori_loop(0, CHUNK, _row, 0)

    _fill(z)
    for k in range(RPT // CHUNK):
        pltpu.sync_copy(ones_v, acc_sh.at[pl.ds(s * RPT + k * CHUNK, CHUNK)])
    _fill(one)
    plsc.subcore_barrier()

    def _group(q, carry):
        pltpu.sync_copy(dst_hbm.at[c, s, pl.ds(q * G, G)], dst_g)
        for b in range(G):
            pltpu.sync_copy(ones_v, acc_sh.at[dst_g.at[b]], add=True)
        return carry
    lax.fori_loop(0, GROUPS, _group, 0)
    plsc.subcore_barrier()

    pltpu.sync_copy(acc_sh.at[pl.ds(s * RPT, RPT)],
                    out_hbm.at[c, pl.ds(s * RPT, RPT)])


def _proj_body(x_ref, wT_ref, b_ref, inp0_ref, h_ref):
    t = jnp.dot(x_ref[...], wT_ref[...], preferred_element_type=jnp.float32)
    t = t + b_ref[...]
    inp0_ref[...] = t
    h_ref[...] = jnp.maximum(t, 0.0)


_tc_proj = pl.pallas_call(
    _proj_body,
    grid=(GRID,),
    in_specs=[
        pl.BlockSpec((BR, D), lambda i: (i, 0)),
        pl.BlockSpec((D, D), lambda i: (0, 0)),
        pl.BlockSpec((1, D), lambda i: (0, 0)),
    ],
    out_specs=[pl.BlockSpec((BR, D), lambda i: (i, 0))] * 2,
    out_shape=[jax.ShapeDtypeStruct((NN, D), jnp.float32)] * 2,
)


def _layer_body(last, p_ref, c_ref, h_ref, inp0_ref, wlT_ref, bl_ref, wrT_ref,
                o_ref):
    p = p_ref[0] + p_ref[1]
    cnt = c_ref[0, :, 0:1] + c_ref[1, :, 0:1]
    mean = p / jnp.maximum(cnt, 1.0)
    acc = jnp.dot(mean, wlT_ref[...], preferred_element_type=jnp.float32)
    acc = acc + bl_ref[...]
    acc = acc + jnp.dot(h_ref[...], wrT_ref[...],
                        preferred_element_type=jnp.float32)
    if last:
        m = jnp.max(acc, axis=-1, keepdims=True)
        lse = jnp.log(jnp.sum(jnp.exp(acc - m), axis=-1, keepdims=True)) + m
        o_ref[...] = acc - lse
    else:
        o_ref[...] = jnp.maximum(acc, 0.0) + 0.2 * inp0_ref[...]


def _make_tc_layer(last):
    return pl.pallas_call(
        functools.partial(_layer_body, last),
        grid=(GRID,),
        in_specs=[
            pl.BlockSpec((NSC, BR, D), lambda i: (0, i, 0)),
            pl.BlockSpec((NSC, BR, CW), lambda i: (0, i, 0)),
            pl.BlockSpec((BR, D), lambda i: (i, 0)),
            pl.BlockSpec((BR, D), lambda i: (i, 0)),
            pl.BlockSpec((D, D), lambda i: (0, 0)),
            pl.BlockSpec((1, D), lambda i: (0, 0)),
            pl.BlockSpec((D, D), lambda i: (0, 0)),
        ],
        out_specs=pl.BlockSpec((BR, D), lambda i: (i, 0)),
        out_shape=jax.ShapeDtypeStruct((NN, D), jnp.float32),
    )


_tc_mid = _make_tc_layer(False)
_tc_last = _make_tc_layer(True)


def kernel(x, edge_index, W_in, b_in, Wl0, bl0, Wr0, Wl1, bl1, Wr1,
           Wl2, bl2, Wr2, Wl3, bl3, Wr3):
    pad = EPAD - EE
    pad_src = jnp.arange(pad, dtype=jnp.int32) % NN
    pad_dst = NN + (jnp.arange(pad, dtype=jnp.int32) % (NPAD - NN))
    srcp = jnp.concatenate([edge_index[0], pad_src]).reshape(
        NSC, NTILES, CHUNKS, CHUNK)
    dstp = jnp.concatenate([edge_index[1], pad_dst]).reshape(
        NSC, NTILES, CHUNKS, CHUNK)

    cntP = _sc_cnt(dstp)
    inp0, h = _tc_proj(x, W_in.T, b_in.reshape(1, D))
    layers = [(Wl0, bl0, Wr0), (Wl1, bl1, Wr1), (Wl2, bl2, Wr2),
              (Wl3, bl3, Wr3)]
    for i, (Wl, bl, Wr) in enumerate(layers):
        P = _sc_agg(h, srcp, dstp)
        tc = _tc_last if i == len(layers) - 1 else _tc_mid
        h = tc(P, cntP, h, inp0, Wl.T, bl.reshape(1, D), Wr.T)
    return h


# speedup vs baseline: 11.3145x; 1.0607x over previous
import functools

import jax
import jax.numpy as jnp
from jax import lax
from jax.experimental import pallas as pl
from jax.experimental.pallas import tpu as pltpu
from jax.experimental.pallas import tpu_sc as plsc

NN = 10000
EE = 320000
D = 128
NTILES = 16
NSC = 2
CHUNK = 128
CW = 128
G = 16
CHUNKS = (-(-EE // (NSC * NTILES * CHUNK * G))) * G
GROUPS = CHUNKS // G
EPT = CHUNKS * CHUNK
EPAD = NSC * NTILES * EPT
NPAD = (-(-(NN + 1) // (NTILES * CHUNK))) * (NTILES * CHUNK)
RPT = NPAD // NTILES
BR = 1000
GRID = NN // BR

_mesh = plsc.VectorSubcoreMesh(core_axis_name="c", subcore_axis_name="s")


@functools.partial(
    pl.kernel,
    mesh=_mesh,
    out_type=jax.ShapeDtypeStruct((NSC, NPAD, D), jnp.float32),
    scratch_types=[
        pltpu.VMEM((2, G, CHUNK), jnp.int32),
        pltpu.VMEM((2, G, CHUNK), jnp.int32),
        pltpu.VMEM((2, CHUNK, D), jnp.float32),
        pltpu.VMEM_SHARED((NPAD, D), jnp.float32),
        pltpu.SemaphoreType.DMA,
        pltpu.SemaphoreType.DMA,
        pltpu.SemaphoreType.DMA,
        pltpu.SemaphoreType.DMA,
        pltpu.SemaphoreType.DMA,
    ],
)
def _sc_agg(h_hbm, src_hbm, dst_hbm, out_hbm, src_g, dst_g, rows_v,
            acc_sh, gsem0, gsem1, ssem0, ssem1, isem):
    c = lax.axis_index("c")
    s = lax.axis_index("s")
    gsem = (gsem0, gsem1)
    ssem = (ssem0, ssem1)

    def _idx_start(g, slot):
        pltpu.async_copy(src_hbm.at[c, s, pl.ds(g * G, G)], src_g.at[slot],
                         isem)
        pltpu.async_copy(dst_hbm.at[c, s, pl.ds(g * G, G)], dst_g.at[slot],
                         isem)

    def _idx_drain(slot):
        pltpu.make_async_copy(src_hbm.at[c, s, pl.ds(0, G)], src_g.at[slot],
                              isem).wait()
        pltpu.make_async_copy(dst_hbm.at[c, s, pl.ds(0, G)], dst_g.at[slot],
                              isem).wait()

    _idx_start(0, 0)

    z = jnp.zeros((16,), jnp.float32)

    def _zrow(r, carry):
        def _zcol(j, carry2):
            rows_v[0, r, pl.ds(j * 16, 16)] = z
            return carry2
        return lax.fori_loop(0, D // 16, _zcol, carry)
    lax.fori_loop(0, CHUNK, _zrow, 0)
    for k in range(RPT // CHUNK):
        pltpu.sync_copy(rows_v.at[0],
                        acc_sh.at[pl.ds(s * RPT + k * CHUNK, CHUNK)])
    plsc.subcore_barrier()

    def _do_group(g, slot):
        _idx_drain(slot)
        _idx_start(lax.rem(g + 1, GROUPS), (slot + 1) % 2)
        gathers = [None] * G
        scatters = [None] * G
        gathers[0] = pltpu.async_copy(
            h_hbm.at[src_g.at[slot, 0]], rows_v.at[0], gsem[0])
        for b in range(G):
            if b + 1 < G:
                if b >= 1:
                    scatters[b - 1].wait()
                gathers[b + 1] = pltpu.async_copy(
                    h_hbm.at[src_g.at[slot, b + 1]],
                    rows_v.at[(b + 1) % 2], gsem[(b + 1) % 2])
            gathers[b].wait()
            scatters[b] = pltpu.async_copy(
                rows_v.at[b % 2], acc_sh.at[dst_g.at[slot, b]],
                ssem[b % 2], add=True)
        scatters[G - 2].wait()
        scatters[G - 1].wait()

    def _pair(m, carry):
        for qq in range(2):
            _do_group(2 * m + qq, qq)
        return carry
    lax.fori_loop(0, GROUPS // 2, _pair, 0)
    if GROUPS % 2:
        _do_group(GROUPS - 1, 0)
    _idx_drain(GROUPS % 2)
    plsc.subcore_barrier()

    pltpu.sync_copy(acc_sh.at[pl.ds(s * RPT, RPT)],
                    out_hbm.at[c, pl.ds(s * RPT, RPT)])


@functools.partial(
    pl.kernel,
    mesh=_mesh,
    out_type=jax.ShapeDtypeStruct((NSC, NPAD, CW), jnp.float32),
    scratch_types=[
        pltpu.VMEM((2, G, CHUNK), jnp.int32),
        pltpu.VMEM((CHUNK, CW), jnp.float32),
        pltpu.VMEM_SHARED((NPAD, CW), jnp.float32),
        pltpu.SemaphoreType.DMA,
        pltpu.SemaphoreType.DMA,
    ],
)
def _sc_cnt(dst_hbm, out_hbm, dst_g, ones_v, acc_sh, ssem, isem):
    c = lax.axis_index("c")
    s = lax.axis_index("s")

    def _idx_start(g, slot):
        pltpu.async_copy(dst_hbm.at[c, s, pl.ds(g * G, G)], dst_g.at[slot],
                         isem)

    def _idx_drain(slot):
        pltpu.make_async_copy(dst_hbm.at[c, s, pl.ds(0, G)], dst_g.at[slot],
                              isem).wait()

    _idx_start(0, 0)

    one = jnp.full((16,), 1.0, jnp.float32)
    z = jnp.zeros((16,), jnp.float32)

    def _fill(val):
        def _row(r, carry):
            def _col(j, carry2):
                ones_v[r, pl.ds(j * 16, 16)] = val
                return carry2
            return lax.fori_loop(0, CW // 16, _col, carry)
        lax.fori_loop(0, CHUNK, _row, 0)

    _fill(z)
    for k in range(RPT // CHUNK):
        pltpu.sync_copy(ones_v, acc_sh.at[pl.ds(s * RPT + k * CHUNK, CHUNK)])
    _fill(one)
    plsc.subcore_barrier()

    def _do_groupc(g, slot):
        _idx_drain(slot)
        _idx_start(lax.rem(g + 1, GROUPS), (slot + 1) % 2)
        scatters = [
            pltpu.async_copy(ones_v, acc_sh.at[dst_g.at[slot, b]],
                             ssem, add=True)
            for b in range(G)
        ]
        for sc in scatters:
            sc.wait()

    def _pairc(m, carry):
        for qq in range(2):
            _do_groupc(2 * m + qq, qq)
        return carry
    lax.fori_loop(0, GROUPS // 2, _pairc, 0)
    if GROUPS % 2:
        _do_groupc(GROUPS - 1, 0)
    _idx_drain(GROUPS % 2)
    plsc.subcore_barrier()

    pltpu.sync_copy(acc_sh.at[pl.ds(s * RPT, RPT)],
                    out_hbm.at[c, pl.ds(s * RPT, RPT)])


def _proj_body(x_ref, wT_ref, b_ref, inp0_ref, h_ref):
    t = jnp.dot(x_ref[...], wT_ref[...], preferred_element_type=jnp.float32)
    t = t + b_ref[...]
    inp0_ref[...] = t
    h_ref[...] = jnp.maximum(t, 0.0)


_tc_proj = pl.pallas_call(
    _proj_body,
    grid=(GRID,),
    in_specs=[
        pl.BlockSpec((BR, D), lambda i: (i, 0)),
        pl.BlockSpec((D, D), lambda i: (0, 0)),
        pl.BlockSpec((1, D), lambda i: (0, 0)),
    ],
    out_specs=[pl.BlockSpec((BR, D), lambda i: (i, 0))] * 2,
    out_shape=[jax.ShapeDtypeStruct((NN, D), jnp.float32)] * 2,
)


def _layer_body(last, p_ref, c_ref, h_ref, inp0_ref, wlT_ref, bl_ref, wrT_ref,
                o_ref):
    p = p_ref[0] + p_ref[1]
    cnt = c_ref[0, :, 0:1] + c_ref[1, :, 0:1]
    mean = p / jnp.maximum(cnt, 1.0)
    acc = jnp.dot(mean, wlT_ref[...], preferred_element_type=jnp.float32)
    acc = acc + bl_ref[...]
    acc = acc + jnp.dot(h_ref[...], wrT_ref[...],
                        preferred_element_type=jnp.float32)
    if last:
        m = jnp.max(acc, axis=-1, keepdims=True)
        lse = jnp.log(jnp.sum(jnp.exp(acc - m), axis=-1, keepdims=True)) + m
        o_ref[...] = acc - lse
    else:
        o_ref[...] = jnp.maximum(acc, 0.0) + 0.2 * inp0_ref[...]


def _make_tc_layer(last):
    return pl.pallas_call(
        functools.partial(_layer_body, last),
        grid=(GRID,),
        in_specs=[
            pl.BlockSpec((NSC, BR, D), lambda i: (0, i, 0)),
            pl.BlockSpec((NSC, BR, CW), lambda i: (0, i, 0)),
            pl.BlockSpec((BR, D), lambda i: (i, 0)),
            pl.BlockSpec((BR, D), lambda i: (i, 0)),
            pl.BlockSpec((D, D), lambda i: (0, 0)),
            pl.BlockSpec((1, D), lambda i: (0, 0)),
            pl.BlockSpec((D, D), lambda i: (0, 0)),
        ],
        out_specs=pl.BlockSpec((BR, D), lambda i: (i, 0)),
        out_shape=jax.ShapeDtypeStruct((NN, D), jnp.float32),
    )


_tc_mid = _make_tc_layer(False)
_tc_last = _make_tc_layer(True)


def kernel(x, edge_index, W_in, b_in, Wl0, bl0, Wr0, Wl1, bl1, Wr1,
           Wl2, bl2, Wr2, Wl3, bl3, Wr3):
    pad = EPAD - EE
    pad_src = jnp.arange(pad, dtype=jnp.int32) % NN
    pad_dst = NN + (jnp.arange(pad, dtype=jnp.int32) % (NPAD - NN))
    srcp = jnp.concatenate([edge_index[0], pad_src]).reshape(
        NSC, NTILES, CHUNKS, CHUNK)
    dstp = jnp.concatenate([edge_index[1], pad_dst]).reshape(
        NSC, NTILES, CHUNKS, CHUNK)

    cntP = _sc_cnt(dstp)
    inp0, h = _tc_proj(x, W_in.T, b_in.reshape(1, D))
    layers = [(Wl0, bl0, Wr0), (Wl1, bl1, Wr1), (Wl2, bl2, Wr2),
              (Wl3, bl3, Wr3)]
    for i, (Wl, bl, Wr) in enumerate(layers):
        P = _sc_agg(h, srcp, dstp)
        tc = _tc_last if i == len(layers) - 1 else _tc_mid
        h = tc(P, cntP, h, inp0, Wl.T, bl.reshape(1, D), Wr.T)
    return h
